# baseline (speedup 1.0000x reference)
import functools

import jax
import jax.numpy as jnp
from jax import lax
from jax.experimental import pallas as pl
from jax.experimental.pallas import tpu as pltpu
from jax.experimental.pallas import tpu_sc as plsc

NUM_CORES = 2
NUM_SUBCORES = 16
NW = NUM_CORES * NUM_SUBCORES
LANES = 16

GROUPS_PER_STEP = 16


def _make_kernel(T, D, B, V):
    assert D % NW == 0
    d_per_w = D // NW
    assert B % (LANES * GROUPS_PER_STEP) == 0
    n_steps = B // (LANES * GROUPS_PER_STEP)
    mesh = plsc.VectorSubcoreMesh(
        core_axis_name="c", subcore_axis_name="s",
        num_cores=NUM_CORES, num_subcores=NUM_SUBCORES)

    @functools.partial(
        pl.kernel,
        out_type=jax.ShapeDtypeStruct((T, D, B), jnp.float32),
        mesh=mesh,
        compiler_params=pltpu.CompilerParams(
            use_tc_tiling_on_sc=True, needs_layout_passes=False,
            disable_bounds_checks=True, disable_semaphore_checks=True),
        scratch_types=[
            pltpu.VMEM((V,), jnp.float32),
            pltpu.VMEM((2, B), jnp.int32),
            pltpu.VMEM((2, B), jnp.float32),
            pltpu.VMEM_SHARED((((T + 7) // 8) * 8, B), jnp.int32),
            pltpu.SemaphoreType.DMA((2,)),
            pltpu.SemaphoreType.DMA((2,)),
            pltpu.SemaphoreType.DMA,
        ],
    )
    def emb(idx_hbm, table_hbm, out_hbm, row_v, idx_v, out_v, idx_sh,
            isems, osems, rsem):
        sid = lax.axis_index("s")
        wid = sid * NUM_CORES + lax.axis_index("c")

        first_row = pltpu.make_async_copy(
            table_hbm.at[wid * d_per_w], row_v, rsem)
        first_row.start()

        n_full = (T // 8) * 8

        @pl.when(sid < 6)
        def _():
            r0 = sid * 8
            pltpu.sync_copy(idx_hbm.at[pl.ds(r0, 8)],
                            idx_sh.at[pl.ds(r0, 8)])

        @pl.when(sid == 6)
        def _():
            pltpu.sync_copy(idx_hbm.at[pl.ds(n_full, T - n_full)],
                            idx_sh.at[pl.ds(n_full, T - n_full)])

        plsc.subcore_barrier()

        def idx_load(t, slot):
            return pltpu.make_async_copy(
                idx_sh.at[t], idx_v.at[slot], isems.at[slot])

        def out_store(t, d, slot):
            return pltpu.make_async_copy(
                out_v.at[slot], out_hbm.at[t, d], osems.at[slot])

        NG = B // LANES
        PIPE = 8
        SLAG = 2

        def gather_row(slot):
            idx_pend = {}
            val_pend = {}
            for g in range(PIPE):
                idx_pend[g] = idx_v[slot, pl.ds(g * LANES, LANES)]
            for g in range(NG + SLAG):
                if g + PIPE < NG:
                    idx_pend[g + PIPE] = idx_v[
                        slot, pl.ds((g + PIPE) * LANES, LANES)]
                if g < NG:
                    val_pend[g] = plsc.load_gather(row_v, [idx_pend.pop(g)])
                if g >= SLAG:
                    out_v[slot, pl.ds((g - SLAG) * LANES, LANES)] = (
                        val_pend.pop(g - SLAG))

        for di in range(d_per_w):
            d = wid * d_per_w + di
            if di == 0:
                first_row.wait()
            else:
                pltpu.sync_copy(table_hbm.at[d], row_v)
            idx_load(0, 0).start()

            def body(t, carry, d=d):
                slot = lax.rem(t, 2)
                nxt = lax.rem(t + 1, 2)
                idx_load(t, slot).wait()

                @pl.when(t + 1 < T)
                def _():
                    idx_load(t + 1, nxt).start()

                @pl.when(t >= 2)
                def _():
                    out_store(t - 2, d, slot).wait()

                gather_row(slot)
                out_store(t, d, slot).start()
                return carry

            lax.fori_loop(0, T, body, 0)
            out_store(T - 2, d, lax.rem(T - 2, 2)).wait()
            out_store(T - 1, d, lax.rem(T - 1, 2)).wait()

    return emb


def kernel(input, table):
    Bt, H = input.shape
    V, D = table.shape
    idxT = input.T
    tableT = table.T
    outT = _make_kernel(H, D, Bt, V)(idxT, tableT)
    return outT.transpose(2, 0, 1)
